# baseline (speedup 1.0000x reference)
import functools

import jax
import jax.numpy as jnp
from jax import lax
from jax.experimental import pallas as pl
from jax.experimental.pallas import tpu as pltpu
from jax.experimental.pallas import tpu_sc as plsc

N_NOTES = 32768
D_FEAT = 64
N_BEATS = 1024
QPM_PRIMO_IDX = 4
TEMPO_IDX = 26

L = 16
NC = 2
NS = 16
NW = NC * NS
W = 2 * D_FEAT

NOTES_PER_TILE = N_NOTES // NS
SCAN_STEPS = NOTES_PER_TILE // L
BEATS_PER_TILE = N_BEATS // NW
GROUP = 128
SENTINEL = N_NOTES - 1


def _body(x_hbm, bn_hbm, out_hbm,
          bnbuf, prevbuf, head, local, stage, fhalf, ffull, rows, row0, outv,
          shared, sem):
    c = lax.axis_index("c")
    s = lax.axis_index("s")
    wid = c * NS + s
    base = s * NOTES_PER_TILE
    iota = lax.iota(jnp.int32, L)
    zeros = jnp.zeros((L,), jnp.int32)

    pltpu.sync_copy(bn_hbm.at[pl.ds(base, NOTES_PER_TILE)], bnbuf)

    @pl.when(s == 0)
    def _():
        for i in range(W // L):
            prevbuf[pl.ds(i * L, L)] = jnp.full((L,), -1, jnp.int32)

    @pl.when(s > 0)
    def _():
        pltpu.sync_copy(bn_hbm.at[pl.ds(base - W, W)], prevbuf)

    pltpu.sync_copy(bn_hbm.at[pl.ds(0, W)], head)
    bn0 = plsc.load_gather(head, [zeros])

    def init_step(i, carry):
        local[pl.ds(i * L, L)] = jnp.full((L,), SENTINEL, jnp.int32)
        return carry

    lax.fori_loop(0, N_BEATS // L, init_step, 0)

    cur = bnbuf[pl.ds(0, L)]
    prev = plsc.load_gather(bnbuf, [jnp.maximum(iota - 1, 0)])
    first_note = plsc.load_gather(bnbuf, [zeros])
    pred_note = plsc.load_gather(prevbuf, [jnp.full((L,), W - 1, jnp.int32)])
    lead_boundary = (first_note != pred_note) | (s == 0)
    boundary = (cur != prev) | ((iota == 0) & lead_boundary)
    plsc.store_scatter(local, [cur - bn0], base + iota, mask=boundary)

    def scan_step(k, carry):
        kcur = bnbuf[pl.ds(k * L, L)]
        kprev = plsc.load_gather(bnbuf, [k * L + iota - 1])
        plsc.store_scatter(local, [kcur - bn0], base + k * L + iota,
                           mask=kcur != kprev)
        return carry

    lax.fori_loop(1, SCAN_STEPS, scan_step, 0)

    pltpu.sync_copy(local, shared.at[pl.ds(s * N_BEATS, N_BEATS)])
    plsc.subcore_barrier()

    gb = (wid // (GROUP // BEATS_PER_TILE)) * GROUP
    off = (wid % (GROUP // BEATS_PER_TILE)) * BEATS_PER_TILE
    for t in range(NS):
        pltpu.sync_copy(shared.at[pl.ds(t * N_BEATS + gb, GROUP)],
                        stage.at[pl.ds(t * GROUP, GROUP)])
    for j in range(BEATS_PER_TILE // L):
        m = stage[pl.ds(off + j * L, L)]
        for t in range(1, NS):
            m = jnp.minimum(m, stage[pl.ds(t * GROUP + off + j * L, L)])
        ffull[pl.ds(j * L, L)] = m
        fhalf[pl.ds(j * L, L)] = m >> 1

    pltpu.async_copy(x_hbm.at[fhalf], rows, sem).wait()

    pltpu.sync_copy(x_hbm.at[0], row0)
    qpm = plsc.load_gather(row0, [jnp.full((L,), QPM_PRIMO_IDX, jnp.int32)])
    tp0 = plsc.load_gather(row0, [jnp.full((L,), D_FEAT - 2, jnp.int32)])
    tp1 = plsc.load_gather(row0, [jnp.full((L,), D_FEAT - 1, jnp.int32)])

    ch = iota % 4
    pattern = jnp.where(ch == 0, qpm, jnp.where(ch == 1, tp0, tp1))
    for m_i in range(BEATS_PER_TILE * 4 // L):
        outv[pl.ds(m_i * L, L)] = pattern
    for j in range(BEATS_PER_TILE // L):
        parity = ffull[pl.ds(j * L, L)] & 1
        col = TEMPO_IDX + (parity << 6)
        tv = plsc.load_gather(rows, [j * L + iota, col])
        plsc.store_scatter(outv, [iota * 4 + (j * L * 4 + 3)], tv)

    pltpu.sync_copy(outv, out_hbm.at[pl.ds(wid * BEATS_PER_TILE * 4,
                                           BEATS_PER_TILE * 4)])


@functools.partial(
    pl.kernel,
    mesh=plsc.VectorSubcoreMesh(core_axis_name="c", subcore_axis_name="s"),
    compiler_params=pltpu.CompilerParams(needs_layout_passes=False),
    out_type=jax.ShapeDtypeStruct((N_BEATS * 4,), jnp.float32),
    scratch_types=[
        pltpu.VMEM((NOTES_PER_TILE,), jnp.int32),
        pltpu.VMEM((W,), jnp.int32),
        pltpu.VMEM((W,), jnp.int32),
        pltpu.VMEM((N_BEATS,), jnp.int32),
        pltpu.VMEM((NS * GROUP,), jnp.int32),
        pltpu.VMEM((BEATS_PER_TILE,), jnp.int32),
        pltpu.VMEM((BEATS_PER_TILE,), jnp.int32),
        pltpu.VMEM((BEATS_PER_TILE, W), jnp.float32),
        pltpu.VMEM((W,), jnp.float32),
        pltpu.VMEM((BEATS_PER_TILE * 4,), jnp.float32),
        pltpu.VMEM_SHARED((NS * N_BEATS,), jnp.int32),
        pltpu.SemaphoreType.DMA,
    ],
)
def _tempo_vec_selector(x_hbm, bn_hbm, out_hbm, *scratch):
    _body(x_hbm, bn_hbm, out_hbm, *scratch)


def kernel(x, beat_numbers):
    x2 = x.reshape(N_NOTES // 2, W)
    bn = beat_numbers.astype(jnp.int32)
    out = _tempo_vec_selector(x2, bn)
    return out.reshape(1, N_BEATS, 4)


# speedup vs baseline: 1.9201x; 1.9201x over previous
import functools

import jax
import jax.numpy as jnp
from jax import lax
from jax.experimental import pallas as pl
from jax.experimental.pallas import tpu as pltpu
from jax.experimental.pallas import tpu_sc as plsc

N_NOTES = 32768
D_FEAT = 64
N_BEATS = 1024
QPM_PRIMO_IDX = 4
TEMPO_IDX = 26

L = 16
NC = 2
NS = 16
NW = NC * NS
W = 128

NOTES_PER_TILE = N_NOTES // NS
SCAN_STEPS = NOTES_PER_TILE // L
BEATS_PER_TILE = N_BEATS // NW
GROUP = 128
SENTINEL = N_NOTES - 1


def _body(bn_hbm, x26_hbm, scal_hbm, out_hbm,
          bnv, xv, prevbuf, headv, scalv, lidx, lval, sidx, sval, outv,
          shared_idx, shared_val, sem):
    c = lax.axis_index("c")
    s = lax.axis_index("s")
    wid = c * NS + s
    base = s * NOTES_PER_TILE
    iota = lax.iota(jnp.int32, L)
    zeros = jnp.zeros((L,), jnp.int32)

    d1 = pltpu.async_copy(bn_hbm.at[pl.ds(base, NOTES_PER_TILE)], bnv, sem)
    d2 = pltpu.async_copy(x26_hbm.at[pl.ds(base, NOTES_PER_TILE)], xv, sem)
    d3 = pltpu.async_copy(bn_hbm.at[pl.ds(0, W)], headv, sem)
    d4 = pltpu.async_copy(scal_hbm, scalv, sem)

    @pl.when(s == 0)
    def _():
        for i in range(W // L):
            prevbuf[pl.ds(i * L, L)] = jnp.full((L,), -1, jnp.int32)

    @pl.when(s > 0)
    def _():
        pltpu.async_copy(bn_hbm.at[pl.ds(base - W, W)], prevbuf, sem).wait()

    d1.wait()
    d2.wait()
    d3.wait()
    d4.wait()

    bn0 = plsc.load_gather(headv, [zeros])

    fill = plsc.load_gather(xv, [jnp.full((L,), NOTES_PER_TILE - 1,
                                          jnp.int32)])

    def init_step(i, carry):
        lidx[pl.ds(i * L, L)] = jnp.full((L,), SENTINEL, jnp.int32)
        lval[pl.ds(i * L, L)] = fill
        return carry

    lax.fori_loop(0, N_BEATS // L, init_step, 0)

    cur = bnv[pl.ds(0, L)]
    prev = plsc.load_gather(bnv, [jnp.maximum(iota - 1, 0)])
    first_note = plsc.load_gather(bnv, [zeros])
    pred_note = plsc.load_gather(prevbuf, [jnp.full((L,), W - 1, jnp.int32)])
    lead_boundary = (first_note != pred_note) | (s == 0)
    boundary = (cur != prev) | ((iota == 0) & lead_boundary)
    rel = cur - bn0
    plsc.store_scatter(lidx, [rel], base + iota, mask=boundary)
    plsc.store_scatter(lval, [rel], xv[pl.ds(0, L)], mask=boundary)

    def scan_step(k, carry):
        kcur = bnv[pl.ds(k * L, L)]
        kprev = plsc.load_gather(bnv, [k * L + iota - 1])
        kb = kcur != kprev
        krel = kcur - bn0
        plsc.store_scatter(lidx, [krel], base + k * L + iota, mask=kb)
        plsc.store_scatter(lval, [krel], xv[pl.ds(k * L, L)], mask=kb)
        return carry

    lax.fori_loop(1, SCAN_STEPS, scan_step, 0, unroll=4)

    pltpu.sync_copy(lidx, shared_idx.at[pl.ds(s * N_BEATS, N_BEATS)])
    pltpu.sync_copy(lval, shared_val.at[pl.ds(s * N_BEATS, N_BEATS)])
    plsc.subcore_barrier()

    gb = (wid // (GROUP // BEATS_PER_TILE)) * GROUP
    off = (wid % (GROUP // BEATS_PER_TILE)) * BEATS_PER_TILE
    drains = []
    for t in range(NS):
        drains.append(pltpu.async_copy(
            shared_idx.at[pl.ds(t * N_BEATS + gb, GROUP)],
            sidx.at[pl.ds(t * GROUP, GROUP)], sem))
        drains.append(pltpu.async_copy(
            shared_val.at[pl.ds(t * N_BEATS + gb, GROUP)],
            sval.at[pl.ds(t * GROUP, GROUP)], sem))
    for d in drains:
        d.wait()

    qpm = plsc.load_gather(scalv, [jnp.full((L,), QPM_PRIMO_IDX, jnp.int32)])
    tp0 = plsc.load_gather(scalv, [jnp.full((L,), D_FEAT - 2, jnp.int32)])
    tp1 = plsc.load_gather(scalv, [jnp.full((L,), D_FEAT - 1, jnp.int32)])
    ch = iota % 4
    pattern = jnp.where(ch == 0, qpm, jnp.where(ch == 1, tp0, tp1))
    for m_i in range(BEATS_PER_TILE * 4 // L):
        outv[pl.ds(m_i * L, L)] = pattern

    for j in range(BEATS_PER_TILE // L):
        m = sidx[pl.ds(off + j * L, L)]
        v = sval[pl.ds(off + j * L, L)]
        for t in range(1, NS):
            ti = sidx[pl.ds(t * GROUP + off + j * L, L)]
            tv = sval[pl.ds(t * GROUP + off + j * L, L)]
            take = ti <= m
            v = jnp.where(take, tv, v)
            m = jnp.minimum(ti, m)
        plsc.store_scatter(outv, [iota * 4 + (j * L * 4 + 3)], v)

    pltpu.sync_copy(outv, out_hbm.at[pl.ds(wid * BEATS_PER_TILE * 4,
                                           BEATS_PER_TILE * 4)])


@functools.partial(
    pl.kernel,
    mesh=plsc.VectorSubcoreMesh(core_axis_name="c", subcore_axis_name="s"),
    compiler_params=pltpu.CompilerParams(needs_layout_passes=False,
                                         use_tc_tiling_on_sc=False),
    out_type=jax.ShapeDtypeStruct((N_BEATS * 4,), jnp.float32),
    scratch_types=[
        pltpu.VMEM((NOTES_PER_TILE,), jnp.int32),
        pltpu.VMEM((NOTES_PER_TILE,), jnp.float32),
        pltpu.VMEM((W,), jnp.int32),
        pltpu.VMEM((W,), jnp.int32),
        pltpu.VMEM((W,), jnp.float32),
        pltpu.VMEM((N_BEATS,), jnp.int32),
        pltpu.VMEM((N_BEATS,), jnp.float32),
        pltpu.VMEM((NS * GROUP,), jnp.int32),
        pltpu.VMEM((NS * GROUP,), jnp.float32),
        pltpu.VMEM((BEATS_PER_TILE * 4,), jnp.float32),
        pltpu.VMEM_SHARED((NS * N_BEATS,), jnp.int32),
        pltpu.VMEM_SHARED((NS * N_BEATS,), jnp.float32),
        pltpu.SemaphoreType.DMA,
    ],
)
def _tempo_vec_selector(bn_hbm, x26_hbm, scal_hbm, out_hbm, *scratch):
    _body(bn_hbm, x26_hbm, scal_hbm, out_hbm, *scratch)


def kernel(x, beat_numbers):
    bn = beat_numbers.astype(jnp.int32)
    x26 = x[0, :, TEMPO_IDX]
    scal = jnp.tile(x[0, 0, :], 2)
    out = _tempo_vec_selector(bn, x26, scal)
    return out.reshape(1, N_BEATS, 4)


# speedup vs baseline: 1.9640x; 1.0228x over previous
import functools

import jax
import jax.numpy as jnp
from jax import lax
from jax.experimental import pallas as pl
from jax.experimental.pallas import tpu as pltpu
from jax.experimental.pallas import tpu_sc as plsc

N_NOTES = 32768
D_FEAT = 64
N_BEATS = 1024
QPM_PRIMO_IDX = 4
TEMPO_IDX = 26

L = 16
NC = 2
NS = 16
NW = NC * NS
W = 128

NOTES_PER_TILE = N_NOTES // NS
SCAN_STEPS = NOTES_PER_TILE // L
BEATS_PER_TILE = N_BEATS // NW
GROUP = 128
SENTINEL = N_NOTES - 1


def _body(bn_hbm, feed_hbm, out_hbm,
          bnv, xv, prevbuf, headv, scalv, lidx, lval, sidx, sval, outv,
          shared_idx, shared_val, sem):
    c = lax.axis_index("c")
    s = lax.axis_index("s")
    wid = c * NS + s
    base = s * NOTES_PER_TILE
    iota = lax.iota(jnp.int32, L)
    zeros = jnp.zeros((L,), jnp.int32)

    d1 = pltpu.async_copy(bn_hbm.at[pl.ds(base, NOTES_PER_TILE)], bnv, sem)
    d2 = pltpu.async_copy(feed_hbm.at[pl.ds(base, NOTES_PER_TILE)], xv, sem)
    d3 = pltpu.async_copy(bn_hbm.at[pl.ds(0, W)], headv, sem)
    d4 = pltpu.async_copy(feed_hbm.at[pl.ds(N_NOTES, W)], scalv, sem)

    @pl.when(s == 0)
    def _():
        for i in range(W // L):
            prevbuf[pl.ds(i * L, L)] = jnp.full((L,), -1, jnp.int32)

    @pl.when(s > 0)
    def _():
        pltpu.async_copy(bn_hbm.at[pl.ds(base - W, W)], prevbuf, sem).wait()

    d1.wait()
    d2.wait()
    d3.wait()
    d4.wait()

    bn0 = plsc.load_gather(headv, [zeros])

    fill = plsc.load_gather(xv, [jnp.full((L,), NOTES_PER_TILE - 1,
                                          jnp.int32)])

    def init_step(i, carry):
        lidx[pl.ds(i * L, L)] = jnp.full((L,), SENTINEL, jnp.int32)
        lval[pl.ds(i * L, L)] = fill
        return carry

    lax.fori_loop(0, N_BEATS // L, init_step, 0)

    cur = bnv[pl.ds(0, L)]
    prev = plsc.load_gather(bnv, [jnp.maximum(iota - 1, 0)])
    first_note = plsc.load_gather(bnv, [zeros])
    pred_note = plsc.load_gather(prevbuf, [jnp.full((L,), W - 1, jnp.int32)])
    lead_boundary = (first_note != pred_note) | (s == 0)
    boundary = (cur != prev) | ((iota == 0) & lead_boundary)
    rel = cur - bn0
    plsc.store_scatter(lidx, [rel], base + iota, mask=boundary)
    plsc.store_scatter(lval, [rel], xv[pl.ds(0, L)], mask=boundary)

    def scan_step(k, carry):
        kcur = bnv[pl.ds(k * L, L)]
        kprev = plsc.load_gather(bnv, [k * L + iota - 1])
        kb = kcur != kprev
        krel = kcur - bn0
        plsc.store_scatter(lidx, [krel], base + k * L + iota, mask=kb)
        plsc.store_scatter(lval, [krel], xv[pl.ds(k * L, L)], mask=kb)
        return carry

    lax.fori_loop(1, SCAN_STEPS, scan_step, 0, unroll=4)

    pltpu.sync_copy(lidx, shared_idx.at[pl.ds(s * N_BEATS, N_BEATS)])
    pltpu.sync_copy(lval, shared_val.at[pl.ds(s * N_BEATS, N_BEATS)])
    plsc.subcore_barrier()

    gb = (wid // (GROUP // BEATS_PER_TILE)) * GROUP
    off = (wid % (GROUP // BEATS_PER_TILE)) * BEATS_PER_TILE
    drains = []
    for t in range(NS):
        drains.append(pltpu.async_copy(
            shared_idx.at[pl.ds(t * N_BEATS + gb, GROUP)],
            sidx.at[pl.ds(t * GROUP, GROUP)], sem))
        drains.append(pltpu.async_copy(
            shared_val.at[pl.ds(t * N_BEATS + gb, GROUP)],
            sval.at[pl.ds(t * GROUP, GROUP)], sem))
    for d in drains:
        d.wait()

    qpm = plsc.load_gather(scalv, [jnp.full((L,), QPM_PRIMO_IDX, jnp.int32)])
    tp0 = plsc.load_gather(scalv, [jnp.full((L,), D_FEAT - 2, jnp.int32)])
    tp1 = plsc.load_gather(scalv, [jnp.full((L,), D_FEAT - 1, jnp.int32)])
    ch = iota % 4
    pattern = jnp.where(ch == 0, qpm, jnp.where(ch == 1, tp0, tp1))
    for m_i in range(BEATS_PER_TILE * 4 // L):
        outv[pl.ds(m_i * L, L)] = pattern

    for j in range(BEATS_PER_TILE // L):
        m = sidx[pl.ds(off + j * L, L)]
        v = sval[pl.ds(off + j * L, L)]
        for t in range(1, NS):
            ti = sidx[pl.ds(t * GROUP + off + j * L, L)]
            tv = sval[pl.ds(t * GROUP + off + j * L, L)]
            take = ti <= m
            v = jnp.where(take, tv, v)
            m = jnp.minimum(ti, m)
        plsc.store_scatter(outv, [iota * 4 + (j * L * 4 + 3)], v)

    pltpu.sync_copy(outv, out_hbm.at[pl.ds(wid * BEATS_PER_TILE * 4,
                                           BEATS_PER_TILE * 4)])


@functools.partial(
    pl.kernel,
    mesh=plsc.VectorSubcoreMesh(core_axis_name="c", subcore_axis_name="s"),
    compiler_params=pltpu.CompilerParams(needs_layout_passes=False,
                                         use_tc_tiling_on_sc=False),
    out_type=jax.ShapeDtypeStruct((N_BEATS * 4,), jnp.float32),
    scratch_types=[
        pltpu.VMEM((NOTES_PER_TILE,), jnp.int32),
        pltpu.VMEM((NOTES_PER_TILE,), jnp.float32),
        pltpu.VMEM((W,), jnp.int32),
        pltpu.VMEM((W,), jnp.int32),
        pltpu.VMEM((W,), jnp.float32),
        pltpu.VMEM((N_BEATS,), jnp.int32),
        pltpu.VMEM((N_BEATS,), jnp.float32),
        pltpu.VMEM((NS * GROUP,), jnp.int32),
        pltpu.VMEM((NS * GROUP,), jnp.float32),
        pltpu.VMEM((BEATS_PER_TILE * 4,), jnp.float32),
        pltpu.VMEM_SHARED((NS * N_BEATS,), jnp.int32),
        pltpu.VMEM_SHARED((NS * N_BEATS,), jnp.float32),
        pltpu.SemaphoreType.DMA,
    ],
)
def _tempo_vec_selector(bn_hbm, feed_hbm, out_hbm, *scratch):
    _body(bn_hbm, feed_hbm, out_hbm, *scratch)


def kernel(x, beat_numbers):
    bn = beat_numbers.astype(jnp.int32)
    feed = jnp.concatenate([x[0, :, TEMPO_IDX], x[0, 0, :], x[0, 0, :]])
    out = _tempo_vec_selector(bn, feed)
    return out.reshape(1, N_BEATS, 4)


# speedup vs baseline: 2.0833x; 1.0608x over previous
import functools

import jax
import jax.numpy as jnp
from jax import lax
from jax.experimental import pallas as pl
from jax.experimental.pallas import tpu as pltpu
from jax.experimental.pallas import tpu_sc as plsc

N_NOTES = 32768
D_FEAT = 64
N_BEATS = 1024
QPM_PRIMO_IDX = 4
TEMPO_IDX = 26

L = 16
NC = 2
NS = 16
NW = NC * NS
W = 128

NOTES_PER_TILE = N_NOTES // NS
SCAN_STEPS = NOTES_PER_TILE // L
BEATS_PER_TILE = N_BEATS // NW
GROUP = 128
SENTINEL = N_NOTES - 1


def _body(bn_hbm, feed_hbm, out_hbm,
          bnv, xv, prevbuf, headv, scalv, lidx, lval, sidx, sval, outv,
          shared_idx, shared_val, sem):
    c = lax.axis_index("c")
    s = lax.axis_index("s")
    wid = c * NS + s
    base = s * NOTES_PER_TILE
    iota = lax.iota(jnp.int32, L)
    zeros = jnp.zeros((L,), jnp.int32)

    d1 = pltpu.async_copy(bn_hbm.at[pl.ds(base, NOTES_PER_TILE)], bnv, sem)
    d2 = pltpu.async_copy(feed_hbm.at[pl.ds(base, NOTES_PER_TILE)], xv, sem)
    d3 = pltpu.async_copy(bn_hbm.at[pl.ds(0, W)], headv, sem)
    d4 = pltpu.async_copy(feed_hbm.at[pl.ds(N_NOTES, W)], scalv, sem)

    @pl.when(s == 0)
    def _():
        for i in range(W // L):
            prevbuf[pl.ds(i * L, L)] = jnp.full((L,), -1, jnp.int32)

    @pl.when(s > 0)
    def _():
        pltpu.async_copy(bn_hbm.at[pl.ds(base - W, W)], prevbuf, sem).wait()

    d1.wait()
    d2.wait()
    d3.wait()
    d4.wait()

    bn0 = plsc.load_gather(headv, [zeros])

    @plsc.parallel_loop(0, N_BEATS // L, unroll=4)
    def _(i):
        lidx[pl.ds(i * L, L)] = jnp.full((L,), SENTINEL, jnp.int32)

    @pl.when(s == NS - 1)
    def _():
        fill = plsc.load_gather(xv, [jnp.full((L,), NOTES_PER_TILE - 1,
                                              jnp.int32)])

        @plsc.parallel_loop(0, N_BEATS // L, unroll=4)
        def _(i):
            lval[pl.ds(i * L, L)] = fill

    cur = bnv[pl.ds(0, L)]
    prev = plsc.load_gather(bnv, [jnp.maximum(iota - 1, 0)])
    first_note = plsc.load_gather(bnv, [zeros])
    pred_note = plsc.load_gather(prevbuf, [jnp.full((L,), W - 1, jnp.int32)])
    lead_boundary = (first_note != pred_note) | (s == 0)
    boundary = (cur != prev) | ((iota == 0) & lead_boundary)
    rel = cur - bn0
    plsc.store_scatter(lidx, [rel], base + iota, mask=boundary)
    plsc.store_scatter(lval, [rel], xv[pl.ds(0, L)], mask=boundary)

    @plsc.parallel_loop(1, SCAN_STEPS, unroll=4)
    def _(k):
        kcur = bnv[pl.ds(k * L, L)]
        kprev = plsc.load_gather(bnv, [k * L + iota - 1])
        kb = kcur != kprev
        krel = kcur - bn0
        plsc.store_scatter(lidx, [krel], base + k * L + iota, mask=kb)
        plsc.store_scatter(lval, [krel], xv[pl.ds(k * L, L)], mask=kb)

    p1 = pltpu.async_copy(lidx, shared_idx.at[pl.ds(s * N_BEATS, N_BEATS)],
                          sem)
    p2 = pltpu.async_copy(lval, shared_val.at[pl.ds(s * N_BEATS, N_BEATS)],
                          sem)
    p1.wait()
    p2.wait()
    plsc.subcore_barrier()

    gb = (wid // (GROUP // BEATS_PER_TILE)) * GROUP
    off = (wid % (GROUP // BEATS_PER_TILE)) * BEATS_PER_TILE
    drains = []
    for t in range(NS):
        drains.append(pltpu.async_copy(
            shared_idx.at[pl.ds(t * N_BEATS + gb, GROUP)],
            sidx.at[pl.ds(t * GROUP, GROUP)], sem))
        drains.append(pltpu.async_copy(
            shared_val.at[pl.ds(t * N_BEATS + gb, GROUP)],
            sval.at[pl.ds(t * GROUP, GROUP)], sem))
    for d in drains:
        d.wait()

    qpm = plsc.load_gather(scalv, [jnp.full((L,), QPM_PRIMO_IDX, jnp.int32)])
    tp0 = plsc.load_gather(scalv, [jnp.full((L,), D_FEAT - 2, jnp.int32)])
    tp1 = plsc.load_gather(scalv, [jnp.full((L,), D_FEAT - 1, jnp.int32)])
    ch = iota % 4
    pattern = jnp.where(ch == 0, qpm, jnp.where(ch == 1, tp0, tp1))
    for m_i in range(BEATS_PER_TILE * 4 // L):
        outv[pl.ds(m_i * L, L)] = pattern

    for j in range(BEATS_PER_TILE // L):
        m = sidx[pl.ds(off + j * L, L)]
        v = sval[pl.ds(off + j * L, L)]
        for t in range(1, NS):
            ti = sidx[pl.ds(t * GROUP + off + j * L, L)]
            tv = sval[pl.ds(t * GROUP + off + j * L, L)]
            take = ti <= m
            v = jnp.where(take, tv, v)
            m = jnp.minimum(ti, m)
        plsc.store_scatter(outv, [iota * 4 + (j * L * 4 + 3)], v)

    pltpu.sync_copy(outv, out_hbm.at[pl.ds(wid * BEATS_PER_TILE * 4,
                                           BEATS_PER_TILE * 4)])


@functools.partial(
    pl.kernel,
    mesh=plsc.VectorSubcoreMesh(core_axis_name="c", subcore_axis_name="s"),
    compiler_params=pltpu.CompilerParams(needs_layout_passes=False,
                                         use_tc_tiling_on_sc=False),
    out_type=jax.ShapeDtypeStruct((N_BEATS * 4,), jnp.float32),
    scratch_types=[
        pltpu.VMEM((NOTES_PER_TILE,), jnp.int32),
        pltpu.VMEM((NOTES_PER_TILE,), jnp.float32),
        pltpu.VMEM((W,), jnp.int32),
        pltpu.VMEM((W,), jnp.int32),
        pltpu.VMEM((W,), jnp.float32),
        pltpu.VMEM((N_BEATS,), jnp.int32),
        pltpu.VMEM((N_BEATS,), jnp.float32),
        pltpu.VMEM((NS * GROUP,), jnp.int32),
        pltpu.VMEM((NS * GROUP,), jnp.float32),
        pltpu.VMEM((BEATS_PER_TILE * 4,), jnp.float32),
        pltpu.VMEM_SHARED((NS * N_BEATS,), jnp.int32),
        pltpu.VMEM_SHARED((NS * N_BEATS,), jnp.float32),
        pltpu.SemaphoreType.DMA,
    ],
)
def _tempo_vec_selector(bn_hbm, feed_hbm, out_hbm, *scratch):
    _body(bn_hbm, feed_hbm, out_hbm, *scratch)


def kernel(x, beat_numbers):
    bn = beat_numbers.astype(jnp.int32)
    feed = jnp.concatenate([x[0, :, TEMPO_IDX], x[0, 0, :], x[0, 0, :]])
    out = _tempo_vec_selector(bn, feed)
    return out.reshape(1, N_BEATS, 4)
